# baseline (speedup 1.0000x reference)
import functools

import jax
import jax.numpy as jnp
from jax import lax
from jax.experimental import pallas as pl
from jax.experimental.pallas import tpu as pltpu
from jax.experimental.pallas import tpu_sc as plsc

N_PIN = 100000
N_NET = 20000
E_PN = 400000
E_NN = 320000
HID = 128
NL = 3

NBP = E_PN // 128
NBN = E_NN // 128

ACC_NET = 20480


def _make_sc_segsum(nb, k_rows, n_src, acc_rows, out_rows, n_passes,
                    pass_stride, in_range, weighted):
    mesh = plsc.VectorSubcoreMesh(core_axis_name="c", subcore_axis_name="s")
    tile_rows = acc_rows // 16
    clamp = n_passes > 1

    def body(tbl, blks, zeros, out, acc,
             blk0, blk1, gid0, gid1, sid0, sid1, row0, row1, sem0, sem1):
        c = lax.axis_index("c")
        s = lax.axis_index("s")
        half = c * n_src
        bufs = ((blk0, gid0, sid0, row0, sem0),
                (blk1, gid1, sid1, row1, sem1))

        def start(bi, b):
            blkbuf, gidx, _, rows, sem = bufs[b]
            pltpu.sync_copy(blks.at[bi], blkbuf)
            for j in range(8):
                v = blkbuf[0, pl.ds(j * 16, 16)]
                gidx[pl.ds(j * 16, 16)] = v + half
            pltpu.async_copy(tbl.at[gidx], rows, sem)

        def finish(base, b):
            blkbuf, gidx, sidx, rows, sem = bufs[b]
            pltpu.make_async_copy(tbl.at[gidx], rows, sem).wait()
            if weighted:
                def mulw(e, cw):
                    wi = plsc.load_gather(
                        blkbuf,
                        [jnp.full((16,), 2, jnp.int32),
                         jnp.full((16,), e, jnp.int32)])
                    w = plsc.bitcast(wi, jnp.float32)
                    for g in range(4):
                        rows[e, pl.ds(g * 16, 16)] = (
                            rows[e, pl.ds(g * 16, 16)] * w)
                    return cw
                lax.fori_loop(0, 128, mulw, 0)
            if clamp:
                for j in range(8):
                    d = blkbuf[1, pl.ds(j * 16, 16)] - base
                    ok = (d >= 0) & (d < in_range)
                    spread = in_range + 88 + (d & 255)
                    sidx[pl.ds(j * 16, 16)] = jnp.where(ok, d, spread)
                pltpu.sync_copy(rows, acc.at[sidx], add=True)
            else:
                pltpu.sync_copy(rows, acc.at[blkbuf.at[1]], add=True)

        def scan_blocks(p):
            base = p * pass_stride
            nbl = (nb - s + 15) // 16
            start(s, 0)

            def pair(j, carry):
                start(s + (2 * j + 1) * 16, 1)
                finish(base, 0)
                pl.when(2 * j + 2 < nbl)(
                    lambda: start(s + (2 * j + 2) * 16, 0))
                finish(base, 1)
                return carry

            lax.fori_loop(0, nbl // 2, pair, 0)
            pl.when(nbl % 2 == 1)(lambda: finish(base, 0))

        for p in range(n_passes):
            pltpu.sync_copy(zeros, acc.at[pl.ds(s * tile_rows, tile_rows)])
            plsc.subcore_barrier()
            scan_blocks(p)
            plsc.subcore_barrier()
            pltpu.sync_copy(
                acc.at[pl.ds(s * tile_rows, tile_rows)],
                out.at[pl.ds(c * out_rows + p * pass_stride + s * tile_rows,
                             tile_rows)])
            if p < n_passes - 1:
                plsc.subcore_barrier()

    return functools.partial(
        pl.kernel,
        out_type=jax.ShapeDtypeStruct((2 * out_rows, 64), jnp.float32),
        mesh=mesh,
        compiler_params=pltpu.CompilerParams(use_tc_tiling_on_sc=False,
                                             needs_layout_passes=False),
        scratch_types=[
            pltpu.VMEM_SHARED((acc_rows, 64), jnp.float32),
            pltpu.VMEM((k_rows, 128), jnp.int32),
            pltpu.VMEM((k_rows, 128), jnp.int32),
            pltpu.VMEM((128,), jnp.int32),
            pltpu.VMEM((128,), jnp.int32),
            pltpu.VMEM((128,), jnp.int32),
            pltpu.VMEM((128,), jnp.int32),
            pltpu.VMEM((128, 64), jnp.float32),
            pltpu.VMEM((128, 64), jnp.float32),
            pltpu.SemaphoreType.DMA,
            pltpu.SemaphoreType.DMA,
        ],
    )(body)


_sc_p2n = _make_sc_segsum(NBP, 2, N_PIN, ACC_NET, ACC_NET, 1, 0, N_NET, False)
_sc_n2n = _make_sc_segsum(NBN, 3, N_NET, ACC_NET, ACC_NET, 1, 0, N_NET, True)

ACC_Q = 50432
HALF_PIN = 50000
OUT_Q = 100608


def _make_sc_n2p_quarter(nb, n_src):
    mesh = plsc.VectorSubcoreMesh(core_axis_name="c", subcore_axis_name="s")
    tile_rows = ACC_Q // 16

    def body(tbl, blks, zeros, out, acc,
             blk0, blk1, gid0, gid1, sid0, sid1, row0, row1, sem0, sem1):
        c = lax.axis_index("c")
        s = lax.axis_index("s")
        bufs = ((blk0, gid0, sid0, row0, sem0),
                (blk1, gid1, sid1, row1, sem1))

        def start(bi, b, qbase):
            blkbuf, gidx, _, rows, sem = bufs[b]
            pltpu.sync_copy(blks.at[bi], blkbuf)
            for j in range(8):
                v = blkbuf[0, pl.ds(j * 16, 16)]
                gidx[pl.ds(j * 16, 16)] = v + qbase
            pltpu.async_copy(tbl.at[gidx], rows, sem)

        def finish(hbase, b):
            blkbuf, gidx, sidx, rows, sem = bufs[b]
            pltpu.make_async_copy(tbl.at[gidx], rows, sem).wait()
            for j in range(8):
                d = blkbuf[1, pl.ds(j * 16, 16)] - hbase
                ok = (d >= 0) & (d < HALF_PIN)
                spread = HALF_PIN + 64 + (d & 255)
                sidx[pl.ds(j * 16, 16)] = jnp.where(ok, d, spread)
            pltpu.sync_copy(rows, acc.at[sidx], add=True)

        nbl = (nb - s + 15) // 16
        for k in range(2):
            qbase = (2 * c + k) * n_src
            for h in range(2):
                hbase = h * HALF_PIN
                pltpu.sync_copy(zeros,
                                acc.at[pl.ds(s * tile_rows, tile_rows)])
                plsc.subcore_barrier()
                start(s, 0, qbase)

                def pair(j, carry):
                    start(s + (2 * j + 1) * 16, 1, qbase)
                    finish(hbase, 0)
                    pl.when(2 * j + 2 < nbl)(
                        lambda: start(s + (2 * j + 2) * 16, 0, qbase))
                    finish(hbase, 1)
                    return carry

                lax.fori_loop(0, nbl // 2, pair, 0)
                pl.when(nbl % 2 == 1)(lambda: finish(hbase, 0))
                plsc.subcore_barrier()
                pltpu.sync_copy(
                    acc.at[pl.ds(s * tile_rows, tile_rows)],
                    out.at[pl.ds((2 * c + k) * OUT_Q + hbase + s * tile_rows,
                                 tile_rows)])
                if not (k == 1 and h == 1):
                    plsc.subcore_barrier()

    return functools.partial(
        pl.kernel,
        out_type=jax.ShapeDtypeStruct((4 * OUT_Q, 32), jnp.float32),
        mesh=mesh,
        compiler_params=pltpu.CompilerParams(use_tc_tiling_on_sc=False,
                                             needs_layout_passes=False),
        scratch_types=[
            pltpu.VMEM_SHARED((ACC_Q, 32), jnp.float32),
            pltpu.VMEM((2, 128), jnp.int32),
            pltpu.VMEM((2, 128), jnp.int32),
            pltpu.VMEM((128,), jnp.int32),
            pltpu.VMEM((128,), jnp.int32),
            pltpu.VMEM((128,), jnp.int32),
            pltpu.VMEM((128,), jnp.int32),
            pltpu.VMEM((128, 32), jnp.float32),
            pltpu.VMEM((128, 32), jnp.float32),
            pltpu.SemaphoreType.DMA,
            pltpu.SemaphoreType.DMA,
        ],
    )(body)


_sc_n2p = _make_sc_n2p_quarter(NBP, N_NET)


def _quarters(w):
    return jnp.stack([w[:64, :64], w[:64, 64:], w[64:, :64], w[64:, 64:]])


def _halves(v):
    return jnp.stack([v[:64], v[64:]])


def _proj_body(x_ref, w_ref, b_ref, o_ref):
    x = x_ref[...]
    for h in range(2):
        o_ref[h] = jax.nn.relu(
            jnp.dot(x, w_ref[h], preferred_element_type=jnp.float32)
            + b_ref[h])


def _proj_split(x, w, b, blk=2000):
    n = x.shape[0]
    return pl.pallas_call(
        _proj_body,
        grid=(n // blk,),
        in_specs=[
            pl.BlockSpec((blk, x.shape[1]), lambda i: (i, 0)),
            pl.BlockSpec((2, x.shape[1], 64), lambda i: (0, 0, 0)),
            pl.BlockSpec((2, 64), lambda i: (0, 0)),
        ],
        out_specs=pl.BlockSpec((2, blk, 64), lambda i: (0, i, 0)),
        out_shape=jax.ShapeDtypeStruct((2, n, 64), jnp.float32),
    )(x, jnp.stack([w[:, :64], w[:, 64:]]), _halves(b))


def _update_body(a_ref, wq_ref, inv_ref, b_ref, r_ref, g_ref, bb_ref, o_ref):
    a0, a1 = a_ref[0], a_ref[1]
    d0 = (jnp.dot(a0, wq_ref[0], preferred_element_type=jnp.float32)
          + jnp.dot(a1, wq_ref[2], preferred_element_type=jnp.float32))
    d1 = (jnp.dot(a0, wq_ref[1], preferred_element_type=jnp.float32)
          + jnp.dot(a1, wq_ref[3], preferred_element_type=jnp.float32))
    inv = inv_ref[...]
    t0 = d0 * inv + b_ref[0] + r_ref[0]
    t1 = d1 * inv + b_ref[1] + r_ref[1]
    mu = (jnp.sum(t0, -1, keepdims=True)
          + jnp.sum(t1, -1, keepdims=True)) * (1.0 / HID)
    var = (jnp.sum((t0 - mu) ** 2, -1, keepdims=True)
           + jnp.sum((t1 - mu) ** 2, -1, keepdims=True)) * (1.0 / HID)
    rs = lax.rsqrt(var + 1e-5)
    o_ref[0] = jax.nn.relu((t0 - mu) * rs * g_ref[0] + bb_ref[0])
    o_ref[1] = jax.nn.relu((t1 - mu) * rs * g_ref[1] + bb_ref[1])


def _update_split(agg, w, inv_deg, b, res, g, beta, blk=2000):
    n = res.shape[1]
    return pl.pallas_call(
        _update_body,
        grid=(n // blk,),
        in_specs=[
            pl.BlockSpec((2, blk, 64), lambda i: (0, i, 0)),
            pl.BlockSpec((4, 64, 64), lambda i: (0, 0, 0)),
            pl.BlockSpec((blk, 1), lambda i: (i, 0)),
            pl.BlockSpec((2, 64), lambda i: (0, 0)),
            pl.BlockSpec((2, blk, 64), lambda i: (0, i, 0)),
            pl.BlockSpec((2, 64), lambda i: (0, 0)),
            pl.BlockSpec((2, 64), lambda i: (0, 0)),
        ],
        out_specs=pl.BlockSpec((2, blk, 64), lambda i: (0, i, 0)),
        out_shape=jax.ShapeDtypeStruct((2, n, 64), jnp.float32),
    )(agg, _quarters(w), inv_deg.reshape(n, 1), _halves(b), res,
      _halves(g), _halves(beta))


def _pred_body(h_ref, w1_ref, b1_ref, w2_ref, b2_ref, o_ref):
    a0, a1 = h_ref[0], h_ref[1]
    h10 = jax.nn.relu(
        jnp.dot(a0, w1_ref[0], preferred_element_type=jnp.float32)
        + jnp.dot(a1, w1_ref[2], preferred_element_type=jnp.float32)
        + b1_ref[0])
    h11 = jax.nn.relu(
        jnp.dot(a0, w1_ref[1], preferred_element_type=jnp.float32)
        + jnp.dot(a1, w1_ref[3], preferred_element_type=jnp.float32)
        + b1_ref[1])
    o_ref[...] = (jnp.dot(h10, w2_ref[0], preferred_element_type=jnp.float32)
                  + jnp.dot(h11, w2_ref[1], preferred_element_type=jnp.float32)
                  + b2_ref[...])


def _predictor_split(h, w1, b1, w2, b2, blk=2000):
    n = h.shape[1]
    out_d = w2.shape[1]
    w2p = jnp.zeros((HID, 64), jnp.float32).at[:, :out_d].set(w2)
    b2p = jnp.zeros((1, 64), jnp.float32).at[0, :out_d].set(b2)
    out = pl.pallas_call(
        _pred_body,
        grid=(n // blk,),
        in_specs=[
            pl.BlockSpec((2, blk, 64), lambda i: (0, i, 0)),
            pl.BlockSpec((4, 64, 64), lambda i: (0, 0, 0)),
            pl.BlockSpec((2, 64), lambda i: (0, 0)),
            pl.BlockSpec((2, 64, 64), lambda i: (0, 0, 0)),
            pl.BlockSpec((1, 64), lambda i: (0, 0)),
        ],
        out_specs=pl.BlockSpec((blk, 64), lambda i: (i, 0)),
        out_shape=jax.ShapeDtypeStruct((n, 64), jnp.float32),
    )(h, _quarters(w1), _halves(b1),
      jnp.stack([w2p[:64], w2p[64:]]), b2p)
    return out[:, :out_d]


def _inv_deg(dst, n_dst):
    deg = jax.ops.segment_sum(jnp.ones(dst.shape, jnp.float32), dst,
                              num_segments=n_dst)
    return 1.0 / jnp.clip(deg, 1.0, None)


def kernel(pin_feats, p2n_src, p2n_dst, n2n_src, n2n_dst, overlap_weights,
           n_net, params):
    inv1 = _inv_deg(p2n_dst, N_NET)
    inv2 = _inv_deg(n2n_dst, N_NET)
    invp = _inv_deg(p2n_src, N_PIN)

    ps = p2n_src.astype(jnp.int32).reshape(NBP, 128)
    pd = p2n_dst.astype(jnp.int32).reshape(NBP, 128)
    blk_p2n = jnp.stack([ps, pd], 1)
    blk_n2p = jnp.stack([pd, ps], 1)
    ns = n2n_src.astype(jnp.int32).reshape(NBN, 128)
    nd = n2n_dst.astype(jnp.int32).reshape(NBN, 128)
    ow = lax.bitcast_convert_type(
        overlap_weights.reshape(NBN, 128), jnp.int32)
    blk_n2n = jnp.stack([ns, nd, ow], 1)

    z_net = jnp.zeros((ACC_NET // 16, 64), jnp.float32)
    z_q = jnp.zeros((ACC_Q // 16, 32), jnp.float32)

    h_pin = _proj_split(pin_feats, params['proj_W'], params['proj_b'])
    h_net = jnp.zeros((2, N_NET, 64), jnp.float32)

    for i in range(NL):
        agg1 = _sc_p2n(h_pin.reshape(2 * N_PIN, 64), blk_p2n, z_net)
        h_net = _update_split(agg1.reshape(2, ACC_NET, 64), params['Wp2n'][i],
                              inv1, params['bp2n'][i], h_net,
                              params['ln1_g'][i], params['ln1_b'][i])
        agg2 = _sc_n2n(h_net.reshape(2 * N_NET, 64), blk_n2n, z_net)
        h_net = _update_split(agg2.reshape(2, ACC_NET, 64), params['Wn2n'][i],
                              inv2, params['bn2n'][i], h_net,
                              params['ln2_g'][i], params['ln2_b'][i])
        if i < NL - 1:
            hq = (h_net.reshape(2, N_NET, 2, 32).transpose(0, 2, 1, 3)
                  .reshape(4 * N_NET, 32))
            agg3q = _sc_n2p(hq, blk_n2p, z_q).reshape(4, OUT_Q, 32)
            agg3 = jnp.stack([
                jnp.concatenate([agg3q[0], agg3q[1]], axis=-1),
                jnp.concatenate([agg3q[2], agg3q[3]], axis=-1)])
            h_pin = _update_split(agg3, params['Wn2p'][i], invp,
                                  params['bn2p'][i], h_pin,
                                  params['lnp_g'][i], params['lnp_b'][i])

    prediction = _predictor_split(h_net, params['pred_W1'], params['pred_b1'],
                                  params['pred_W2'], params['pred_b2'])
    h_net_full = jnp.concatenate([h_net[0], h_net[1]], axis=1)
    return (prediction, h_net_full)


# speedup vs baseline: 1.2039x; 1.2039x over previous
import functools

import jax
import jax.numpy as jnp
from jax import lax
from jax.experimental import pallas as pl
from jax.experimental.pallas import tpu as pltpu
from jax.experimental.pallas import tpu_sc as plsc

N_PIN = 100000
N_NET = 20000
E_PN = 400000
E_NN = 320000
HID = 128
NL = 3

NBP = E_PN // 128
NBN = E_NN // 128

ACC_NET = 20480


def _make_sc_segsum(nb, k_rows, n_src, acc_rows, out_rows, n_passes,
                    pass_stride, in_range, weighted, nbufs=4):
    mesh = plsc.VectorSubcoreMesh(core_axis_name="c", subcore_axis_name="s")
    tile_rows = acc_rows // 16
    clamp = n_passes > 1

    def body(tbl, blks, zeros, out, acc, *scr):
        c = lax.axis_index("c")
        s = lax.axis_index("s")
        half = c * n_src
        bufs = tuple(scr[5 * b:5 * b + 5] for b in range(nbufs))

        def start(bi, b):
            blkbuf, gidx, _, rows, sem = bufs[b]
            pltpu.sync_copy(blks.at[bi], blkbuf)
            for j in range(8):
                v = blkbuf[0, pl.ds(j * 16, 16)]
                gidx[pl.ds(j * 16, 16)] = v + half
            pltpu.async_copy(tbl.at[gidx], rows, sem)

        def finish(base, b):
            blkbuf, gidx, sidx, rows, sem = bufs[b]
            pltpu.make_async_copy(tbl.at[gidx], rows, sem).wait()
            if weighted:
                def mulw(e, cw):
                    wi = plsc.load_gather(
                        blkbuf,
                        [jnp.full((16,), 2, jnp.int32),
                         jnp.full((16,), e, jnp.int32)])
                    w = plsc.bitcast(wi, jnp.float32)
                    for g in range(4):
                        rows[e, pl.ds(g * 16, 16)] = (
                            rows[e, pl.ds(g * 16, 16)] * w)
                    return cw
                lax.fori_loop(0, 128, mulw, 0)
            if clamp:
                for j in range(8):
                    d = blkbuf[1, pl.ds(j * 16, 16)] - base
                    ok = (d >= 0) & (d < in_range)
                    spread = in_range + 88 + (d & 255)
                    sidx[pl.ds(j * 16, 16)] = jnp.where(ok, d, spread)
                pltpu.sync_copy(rows, acc.at[sidx], add=True)
            else:
                pltpu.sync_copy(rows, acc.at[blkbuf.at[1]], add=True)

        def scan_blocks(p):
            base = p * pass_stride
            nbl = (nb - s + 15) // 16
            blkof = lambda j: s + j * 16
            for t in range(nbufs - 1):
                start(blkof(t), t)

            def group(j, carry):
                start(blkof(nbufs * j + nbufs - 1), nbufs - 1)
                finish(base, 0)
                for t in range(nbufs - 1):
                    pl.when(nbufs * j + nbufs + t < nbl)(
                        lambda t=t: start(blkof(nbufs * j + nbufs + t), t))
                    finish(base, t + 1)
                return carry

            lax.fori_loop(0, nbl // nbufs, group, 0)
            for t in range(nbufs - 1):
                pl.when(nbufs * (nbl // nbufs) + t < nbl)(
                    lambda t=t: finish(base, t))

        for p in range(n_passes):
            pltpu.sync_copy(zeros, acc.at[pl.ds(s * tile_rows, tile_rows)])
            plsc.subcore_barrier()
            scan_blocks(p)
            plsc.subcore_barrier()
            pltpu.sync_copy(
                acc.at[pl.ds(s * tile_rows, tile_rows)],
                out.at[pl.ds(c * out_rows + p * pass_stride + s * tile_rows,
                             tile_rows)])
            if p < n_passes - 1:
                plsc.subcore_barrier()

    return functools.partial(
        pl.kernel,
        out_type=jax.ShapeDtypeStruct((2 * out_rows, 64), jnp.float32),
        mesh=mesh,
        compiler_params=pltpu.CompilerParams(use_tc_tiling_on_sc=False,
                                             needs_layout_passes=False),
        scratch_types=[pltpu.VMEM_SHARED((acc_rows, 64), jnp.float32)] + [
            t for _ in range(nbufs) for t in (
                pltpu.VMEM((k_rows, 128), jnp.int32),
                pltpu.VMEM((128,), jnp.int32),
                pltpu.VMEM((128,), jnp.int32),
                pltpu.VMEM((128, 64), jnp.float32),
                pltpu.SemaphoreType.DMA,
            )],
    )(body)


ACC_PIN = 25600
PIN_PASS = 25000
OUT_PIN = 100608

_sc_p2n = _make_sc_segsum(NBP, 2, N_PIN, ACC_NET, ACC_NET, 1, 0, N_NET, False)
_sc_n2n = _make_sc_segsum(NBN, 3, N_NET, ACC_NET, ACC_NET, 1, 0, N_NET, True)
_sc_n2p = _make_sc_segsum(NBP, 2, N_NET, ACC_PIN, OUT_PIN, 4, PIN_PASS,
                          PIN_PASS, False, nbufs=2)


def _quarters(w):
    return jnp.stack([w[:64, :64], w[:64, 64:], w[64:, :64], w[64:, 64:]])


def _halves(v):
    return jnp.stack([v[:64], v[64:]])


def _proj_body(x_ref, w_ref, b_ref, o_ref):
    x = x_ref[...]
    for h in range(2):
        o_ref[h] = jax.nn.relu(
            jnp.dot(x, w_ref[h], preferred_element_type=jnp.float32)
            + b_ref[h])


def _proj_split(x, w, b, blk=2000):
    n = x.shape[0]
    return pl.pallas_call(
        _proj_body,
        grid=(n // blk,),
        in_specs=[
            pl.BlockSpec((blk, x.shape[1]), lambda i: (i, 0)),
            pl.BlockSpec((2, x.shape[1], 64), lambda i: (0, 0, 0)),
            pl.BlockSpec((2, 64), lambda i: (0, 0)),
        ],
        out_specs=pl.BlockSpec((2, blk, 64), lambda i: (0, i, 0)),
        out_shape=jax.ShapeDtypeStruct((2, n, 64), jnp.float32),
    )(x, jnp.stack([w[:, :64], w[:, 64:]]), _halves(b))


def _update_body(a_ref, wq_ref, inv_ref, b_ref, r_ref, g_ref, bb_ref, o_ref):
    a0, a1 = a_ref[0], a_ref[1]
    d0 = (jnp.dot(a0, wq_ref[0], preferred_element_type=jnp.float32)
          + jnp.dot(a1, wq_ref[2], preferred_element_type=jnp.float32))
    d1 = (jnp.dot(a0, wq_ref[1], preferred_element_type=jnp.float32)
          + jnp.dot(a1, wq_ref[3], preferred_element_type=jnp.float32))
    inv = inv_ref[...]
    t0 = d0 * inv + b_ref[0] + r_ref[0]
    t1 = d1 * inv + b_ref[1] + r_ref[1]
    mu = (jnp.sum(t0, -1, keepdims=True)
          + jnp.sum(t1, -1, keepdims=True)) * (1.0 / HID)
    var = (jnp.sum((t0 - mu) ** 2, -1, keepdims=True)
           + jnp.sum((t1 - mu) ** 2, -1, keepdims=True)) * (1.0 / HID)
    rs = lax.rsqrt(var + 1e-5)
    o_ref[0] = jax.nn.relu((t0 - mu) * rs * g_ref[0] + bb_ref[0])
    o_ref[1] = jax.nn.relu((t1 - mu) * rs * g_ref[1] + bb_ref[1])


def _update_split(agg, w, inv_deg, b, res, g, beta, blk=2000):
    n = res.shape[1]
    return pl.pallas_call(
        _update_body,
        grid=(n // blk,),
        in_specs=[
            pl.BlockSpec((2, blk, 64), lambda i: (0, i, 0)),
            pl.BlockSpec((4, 64, 64), lambda i: (0, 0, 0)),
            pl.BlockSpec((blk, 1), lambda i: (i, 0)),
            pl.BlockSpec((2, 64), lambda i: (0, 0)),
            pl.BlockSpec((2, blk, 64), lambda i: (0, i, 0)),
            pl.BlockSpec((2, 64), lambda i: (0, 0)),
            pl.BlockSpec((2, 64), lambda i: (0, 0)),
        ],
        out_specs=pl.BlockSpec((2, blk, 64), lambda i: (0, i, 0)),
        out_shape=jax.ShapeDtypeStruct((2, n, 64), jnp.float32),
    )(agg, _quarters(w), inv_deg.reshape(n, 1), _halves(b), res,
      _halves(g), _halves(beta))


def _pred_body(h_ref, w1_ref, b1_ref, w2_ref, b2_ref, o_ref):
    a0, a1 = h_ref[0], h_ref[1]
    h10 = jax.nn.relu(
        jnp.dot(a0, w1_ref[0], preferred_element_type=jnp.float32)
        + jnp.dot(a1, w1_ref[2], preferred_element_type=jnp.float32)
        + b1_ref[0])
    h11 = jax.nn.relu(
        jnp.dot(a0, w1_ref[1], preferred_element_type=jnp.float32)
        + jnp.dot(a1, w1_ref[3], preferred_element_type=jnp.float32)
        + b1_ref[1])
    o_ref[...] = (jnp.dot(h10, w2_ref[0], preferred_element_type=jnp.float32)
                  + jnp.dot(h11, w2_ref[1], preferred_element_type=jnp.float32)
                  + b2_ref[...])


def _predictor_split(h, w1, b1, w2, b2, blk=2000):
    n = h.shape[1]
    out_d = w2.shape[1]
    w2p = jnp.zeros((HID, 64), jnp.float32).at[:, :out_d].set(w2)
    b2p = jnp.zeros((1, 64), jnp.float32).at[0, :out_d].set(b2)
    out = pl.pallas_call(
        _pred_body,
        grid=(n // blk,),
        in_specs=[
            pl.BlockSpec((2, blk, 64), lambda i: (0, i, 0)),
            pl.BlockSpec((4, 64, 64), lambda i: (0, 0, 0)),
            pl.BlockSpec((2, 64), lambda i: (0, 0)),
            pl.BlockSpec((2, 64, 64), lambda i: (0, 0, 0)),
            pl.BlockSpec((1, 64), lambda i: (0, 0)),
        ],
        out_specs=pl.BlockSpec((blk, 64), lambda i: (i, 0)),
        out_shape=jax.ShapeDtypeStruct((n, 64), jnp.float32),
    )(h, _quarters(w1), _halves(b1),
      jnp.stack([w2p[:64], w2p[64:]]), b2p)
    return out[:, :out_d]


def _inv_deg(dst, n_dst):
    deg = jax.ops.segment_sum(jnp.ones(dst.shape, jnp.float32), dst,
                              num_segments=n_dst)
    return 1.0 / jnp.clip(deg, 1.0, None)


def kernel(pin_feats, p2n_src, p2n_dst, n2n_src, n2n_dst, overlap_weights,
           n_net, params):
    inv1 = _inv_deg(p2n_dst, N_NET)
    inv2 = _inv_deg(n2n_dst, N_NET)
    invp = _inv_deg(p2n_src, N_PIN)

    ps = p2n_src.astype(jnp.int32).reshape(NBP, 128)
    pd = p2n_dst.astype(jnp.int32).reshape(NBP, 128)
    blk_p2n = jnp.stack([ps, pd], 1)
    blk_n2p = jnp.stack([pd, ps], 1)
    ns = n2n_src.astype(jnp.int32).reshape(NBN, 128)
    nd = n2n_dst.astype(jnp.int32).reshape(NBN, 128)
    ow = lax.bitcast_convert_type(
        overlap_weights.reshape(NBN, 128), jnp.int32)
    blk_n2n = jnp.stack([ns, nd, ow], 1)

    z_net = jnp.zeros((ACC_NET // 16, 64), jnp.float32)
    z_pin = jnp.zeros((ACC_PIN // 16, 64), jnp.float32)

    h_pin = _proj_split(pin_feats, params['proj_W'], params['proj_b'])
    h_net = jnp.zeros((2, N_NET, 64), jnp.float32)

    for i in range(NL):
        agg1 = _sc_p2n(h_pin.reshape(2 * N_PIN, 64), blk_p2n, z_net)
        h_net = _update_split(agg1.reshape(2, ACC_NET, 64), params['Wp2n'][i],
                              inv1, params['bp2n'][i], h_net,
                              params['ln1_g'][i], params['ln1_b'][i])
        agg2 = _sc_n2n(h_net.reshape(2 * N_NET, 64), blk_n2n, z_net)
        h_net = _update_split(agg2.reshape(2, ACC_NET, 64), params['Wn2n'][i],
                              inv2, params['bn2n'][i], h_net,
                              params['ln2_g'][i], params['ln2_b'][i])
        if i < NL - 1:
            agg3 = _sc_n2p(h_net.reshape(2 * N_NET, 64), blk_n2p, z_pin)
            h_pin = _update_split(agg3.reshape(2, OUT_PIN, 64),
                                  params['Wn2p'][i], invp, params['bn2p'][i],
                                  h_pin, params['lnp_g'][i],
                                  params['lnp_b'][i])

    prediction = _predictor_split(h_net, params['pred_W1'], params['pred_b1'],
                                  params['pred_W2'], params['pred_b2'])
    h_net_full = jnp.concatenate([h_net[0], h_net[1]], axis=1)
    return (prediction, h_net_full)
